# baseline (speedup 1.0000x reference)
import jax
import jax.numpy as jnp
from jax.experimental import pallas as pl
from jax.experimental.pallas import tpu as pltpu

N = 4096
IN_FEAT = 64
OUT_FEAT = 64
BMAX = 512
SIZES = [512] * 7 + [256, 128, 64, 64]
assert sum(SIZES) == N
OFFS = [sum(SIZES[:i]) for i in range(len(SIZES))]
T = len(SIZES)
D = 6


def _adj_copy(adj_hbm, bufs, sems, t):
    return pltpu.make_async_copy(
        adj_hbm.at[pl.ds(OFFS[t], SIZES[t]), :],
        bufs.at[t % D, pl.ds(0, SIZES[t]), :],
        sems.at[t % D],
    )


def _kern(feat_hbm, w_hbm, adj_hbm, out_hbm, feat_v, w_v, x_ref, out_v,
          bufs, sems, fsem, wsem, osem):
    for t in range(D):
        _adj_copy(adj_hbm, bufs, sems, t).start()
    fcp = pltpu.make_async_copy(feat_hbm, feat_v, fsem)
    wcp = pltpu.make_async_copy(w_hbm, w_v, wsem)
    fcp.start()
    wcp.start()
    fcp.wait()
    wcp.wait()

    x_ref[...] = jnp.dot(
        feat_v[...], w_v[...], preferred_element_type=jnp.float32
    )
    x = x_ref[...]

    for t in range(T):
        _adj_copy(adj_hbm, bufs, sems, t).wait()
        out_v[pl.ds(OFFS[t], SIZES[t]), :] = jnp.dot(
            bufs[t % D, pl.ds(0, SIZES[t]), :], x,
            preferred_element_type=jnp.float32,
        )
        if t + D < T:
            _adj_copy(adj_hbm, bufs, sems, t + D).start()

    ocp = pltpu.make_async_copy(out_v, out_hbm, osem)
    ocp.start()
    ocp.wait()


@jax.jit
def kernel(feat, adj, weight2):
    return pl.pallas_call(
        _kern,
        in_specs=[
            pl.BlockSpec(memory_space=pltpu.HBM),
            pl.BlockSpec(memory_space=pltpu.HBM),
            pl.BlockSpec(memory_space=pltpu.HBM),
        ],
        out_specs=pl.BlockSpec(memory_space=pltpu.HBM),
        out_shape=jax.ShapeDtypeStruct((N, OUT_FEAT), jnp.float32),
        scratch_shapes=[
            pltpu.VMEM((N, IN_FEAT), jnp.float32),
            pltpu.VMEM((IN_FEAT, OUT_FEAT), jnp.float32),
            pltpu.VMEM((N, OUT_FEAT), jnp.float32),
            pltpu.VMEM((N, OUT_FEAT), jnp.float32),
            pltpu.VMEM((D, BMAX, N), jnp.float32),
            pltpu.SemaphoreType.DMA((D,)),
            pltpu.SemaphoreType.DMA,
            pltpu.SemaphoreType.DMA,
            pltpu.SemaphoreType.DMA,
        ],
        compiler_params=pltpu.CompilerParams(
            vmem_limit_bytes=60 * 1024 * 1024,
        ),
    )(feat, weight2, adj)


# speedup vs baseline: 1.4055x; 1.4055x over previous
import jax
import jax.numpy as jnp
from jax.experimental import pallas as pl
from jax.experimental.pallas import tpu as pltpu

N = 4096
IN_FEAT = 64
OUT_FEAT = 64
BM = 512


def _kern(feat_ref, w_ref, adj_ref, out_ref):
    y = jnp.dot(adj_ref[...], feat_ref[...], preferred_element_type=jnp.float32)
    out_ref[...] = jnp.dot(y, w_ref[...], preferred_element_type=jnp.float32)


@jax.jit
def kernel(feat, adj, weight2):
    grid = (N // BM,)
    return pl.pallas_call(
        _kern,
        grid=grid,
        in_specs=[
            pl.BlockSpec((N, IN_FEAT), lambda i: (0, 0)),
            pl.BlockSpec((IN_FEAT, OUT_FEAT), lambda i: (0, 0)),
            pl.BlockSpec((BM, N), lambda i: (i, 0)),
        ],
        out_specs=pl.BlockSpec((BM, OUT_FEAT), lambda i: (i, 0)),
        out_shape=jax.ShapeDtypeStruct((N, OUT_FEAT), jnp.float32),
        compiler_params=pltpu.CompilerParams(
            dimension_semantics=(pltpu.PARALLEL,),
        ),
    )(feat, weight2, adj)


# speedup vs baseline: 1.4162x; 1.0077x over previous
import jax
import jax.numpy as jnp
from jax.experimental import pallas as pl
from jax.experimental.pallas import tpu as pltpu

N = 4096
IN_FEAT = 64
OUT_FEAT = 64
BM = 512
H = N // 2


def _kern(featw_ref, w_ref, adj_ref, out_ref, x_ref):
    @pl.when(pl.program_id(0) == 0)
    def _():
        w = w_ref[...]
        x_ref[:H, :] = jnp.dot(
            featw_ref[:, :IN_FEAT], w, preferred_element_type=jnp.float32
        )
        x_ref[H:, :] = jnp.dot(
            featw_ref[:, IN_FEAT:], w, preferred_element_type=jnp.float32
        )

    out_ref[...] = jnp.dot(
        adj_ref[...], x_ref[...], preferred_element_type=jnp.float32
    )


@jax.jit
def kernel(feat, adj, weight2):
    featw = jnp.concatenate([feat[:H], feat[H:]], axis=1)
    grid = (N // BM,)
    return pl.pallas_call(
        _kern,
        grid=grid,
        in_specs=[
            pl.BlockSpec((H, 2 * IN_FEAT), lambda i: (0, 0)),
            pl.BlockSpec((IN_FEAT, OUT_FEAT), lambda i: (0, 0)),
            pl.BlockSpec((BM, N), lambda i: (i, 0)),
        ],
        out_specs=pl.BlockSpec((BM, OUT_FEAT), lambda i: (i, 0)),
        out_shape=jax.ShapeDtypeStruct((N, OUT_FEAT), jnp.float32),
        scratch_shapes=[pltpu.VMEM((N, OUT_FEAT), jnp.float32)],
    )(featw, weight2, adj)
